# baseline (speedup 1.0000x reference)
import functools

import jax
import jax.numpy as jnp
from jax import lax
from jax.experimental import pallas as pl
from jax.experimental.pallas import tpu as pltpu
from jax.experimental.pallas import tpu_sc as plsc


def _combo_body(tt_ref, st_ref, w_ref, b_ref, out_ref):
    h = tt_ref.shape[1]
    n_t = tt_ref.shape[0]
    n_s = st_ref.shape[0]
    lanes = out_ref.shape[1]
    packs = lanes // h
    rows = out_ref.shape[0] * packs
    tp = jnp.dot(tt_ref[...], w_ref[0:h, :], preferred_element_type=jnp.float32)
    sp = jnp.dot(st_ref[...], w_ref[h:2 * h, :], preferred_element_type=jnp.float32)
    stacked = jnp.concatenate([tp, sp], axis=0)
    pieces = []
    for j in range(packs):
        i = lax.broadcasted_iota(jnp.int32, (rows // packs, n_t + n_s), 0)
        i = i * packs + j
        jj = lax.broadcasted_iota(jnp.int32, (rows // packs, n_t + n_s), 1)
        tsel = (jj < n_t) & (i // n_s == jj)
        ssel = (jj >= n_t) & (i % n_s == jj - n_t)
        sel = jnp.where(tsel | ssel, 1.0, 0.0).astype(jnp.float32)
        pieces.append(
            jnp.dot(sel, stacked, preferred_element_type=jnp.float32)
            + b_ref[...])
    out_ref[...] = jnp.concatenate(pieces, axis=1)


def _build_combo(tier_table, scope_table, W, b):
    h = tier_table.shape[1]
    return pl.pallas_call(
        _combo_body,
        out_shape=jax.ShapeDtypeStruct((8, 128), jnp.float32),
    )(tier_table, scope_table, W, b.reshape(1, h))


def _cidx_body(t_ref, s_ref, out_ref, *, n_scope, h, seq):
    cidx = (t_ref[...] * n_scope + s_ref[...]) * h
    k = t_ref.shape[0] // 8
    pieces = []
    for j in range(k):
        blk = cidx[j * 8:(j + 1) * 8, :]
        pieces.append(blk[:, 0:128])
        pieces.append(jnp.pad(blk[:, 128:seq], ((0, 0), (0, 256 - seq))))
    out_ref[...] = jnp.concatenate(pieces, axis=0)


def _build_cidx(tier_2d, scope_2d, n_scope, h):
    bsz, seq = tier_2d.shape
    k = 32
    grid = bsz // (8 * k)
    body = functools.partial(_cidx_body, n_scope=n_scope, h=h, seq=seq)
    return pl.pallas_call(
        body,
        grid=(grid,),
        compiler_params=pltpu.CompilerParams(
            dimension_semantics=("arbitrary",)),
        in_specs=[
            pl.BlockSpec((8 * k, seq), lambda i: (i, 0)),
            pl.BlockSpec((8 * k, seq), lambda i: (i, 0)),
        ],
        out_specs=pl.BlockSpec((16 * k, 128), lambda i: (i, 0)),
        out_shape=jax.ShapeDtypeStruct((bsz * 2, 128), jnp.int32),
    )(tier_2d, scope_2d)


def _sc_lookup(cidx_2d, combo_2d, bsz, seq, h):
    n_pad = cidx_2d.shape[0] * 128
    info = plsc.get_sparse_core_info()
    nc, ns, nl = info.num_cores, info.num_subcores, info.num_lanes
    nw = nc * ns
    tpr = 2048
    ntr = n_pad // tpr
    g = ntr // nw
    ch_out = 8 * seq * h
    assert g * nw == ntr

    plan = []
    for half in range(2):
        for c0 in range(0, 128, nl):
            c = c0 + 128 * half
            if c >= seq:
                continue
            nv = min(nl, seq - c)
            plan.append((half * 1024 + c0, c * h, nv))

    mesh = plsc.VectorSubcoreMesh(core_axis_name="c", subcore_axis_name="s")

    @functools.partial(
        pl.kernel,
        out_type=jax.ShapeDtypeStruct((bsz, seq * h), jnp.float32),
        mesh=mesh,
        compiler_params=pltpu.CompilerParams(needs_layout_passes=False),
        scratch_types=[
            pltpu.VMEM((1024,), jnp.float32),
            pltpu.VMEM((16, 128), jnp.int32),
            pltpu.VMEM((16, 128), jnp.int32),
            pltpu.VMEM((8, seq * h), jnp.float32),
            pltpu.VMEM((8, seq * h), jnp.float32),
            pltpu.SemaphoreType.DMA,
            pltpu.SemaphoreType.DMA,
            pltpu.SemaphoreType.DMA,
            pltpu.SemaphoreType.DMA,
        ],
    )
    def body(cidx_hbm, combo_hbm, out_hbm,
             combo_v, cidx_v0, cidx_v1, rows_v0, rows_v1,
             si0, si1, so0, so1):
        wid = lax.axis_index("s") * nc + lax.axis_index("c")
        tr_base = wid * g
        for i in range(8):
            pltpu.sync_copy(combo_hbm.at[i], combo_v.at[pl.ds(i * 128, 128)])
        iota = lax.iota(jnp.int32, nl)
        cids = (cidx_v0, cidx_v1)
        rowsb = (rows_v0, rows_v1)
        sis = (si0, si1)
        sos = (so0, so1)

        def issue_in(gi, p):
            pltpu.async_copy(
                cidx_hbm.at[pl.ds((tr_base + gi) * 16, 16)], cids[p], sis[p])

        def compute_half(p, hb):
            def rowloop(rr0, c2):
                rr = rr0 + hb * 4
                for (src_c, dst_c, nv) in plan:
                    cvec = cids[p][rr + src_c // 128, pl.ds(src_c % 128, nl)]
                    for j in range(nv):
                        bj = lax.gather(
                            cvec,
                            jnp.full((nl, 1), j, jnp.int32),
                            lax.GatherDimensionNumbers(
                                offset_dims=(),
                                collapsed_slice_dims=(0,),
                                start_index_map=(0,),
                            ),
                            (1,),
                            mode=lax.GatherScatterMode.PROMISE_IN_BOUNDS,
                        )
                        halves = [
                            plsc.load_gather(combo_v, [bj + iota + q * nl])
                            for q in range(h // nl)
                        ]
                        toff = dst_c + j * h
                        for q in range(h // nl):
                            rowsb[p][rr, pl.ds(toff + q * nl, nl)] = halves[q]
                return c2

            lax.fori_loop(0, 4, rowloop, 0)

        issue_in(0, 0)
        npairs = g // 2

        def pair(k, carry):
            for p in (0, 1):
                gi = k * 2 + p
                pltpu.make_async_copy(
                    cidx_hbm.at[pl.ds(0, 16)], cids[p], sis[p]).wait()

                @pl.when(gi + 1 < g)
                def _():
                    issue_in(gi + 1, 1 - p)

                @pl.when(k >= 1)
                def _():
                    for _hb in (0, 1):
                        pltpu.make_async_copy(
                            rowsb[p].at[pl.ds(0, 4)],
                            out_hbm.at[pl.ds(0, 4)], sos[p]).wait()

                r0 = (tr_base + gi) * 8
                for hb in (0, 1):
                    compute_half(p, hb)
                    pltpu.async_copy(
                        rowsb[p].at[pl.ds(hb * 4, 4)],
                        out_hbm.at[pl.ds(r0 + hb * 4, 4)], sos[p])
            return carry

        lax.fori_loop(0, npairs, pair, 0)
        for p in (0, 1):
            for _hb in (0, 1):
                pltpu.make_async_copy(
                    rowsb[p].at[pl.ds(0, 4)],
                    out_hbm.at[pl.ds(0, 4)], sos[p]).wait()

    return body(cidx_2d, combo_2d)


def kernel(tier_indices, scope_indices, tier_table, scope_table, W, b):
    bsz, seq = tier_indices.shape
    n_scope = scope_table.shape[0]
    h = tier_table.shape[1]
    combo = _build_combo(tier_table, scope_table, W, b)
    tier_i = tier_indices.astype(jnp.int32)
    scope_i = scope_indices.astype(jnp.int32)
    cidx = _build_cidx(tier_i, scope_i, n_scope, h)
    out = _sc_lookup(cidx, combo, bsz, seq, h)
    return out.reshape(bsz, seq, h)


# speedup vs baseline: 1.2936x; 1.2936x over previous
import functools

import jax
import jax.numpy as jnp
from jax import lax
from jax.experimental import pallas as pl
from jax.experimental.pallas import tpu as pltpu
from jax.experimental.pallas import tpu_sc as plsc


def _combo_body(tt_ref, st_ref, w_ref, b_ref, out_ref):
    h = tt_ref.shape[1]
    n_t = tt_ref.shape[0]
    n_s = st_ref.shape[0]
    lanes = out_ref.shape[1]
    packs = lanes // h
    rows = out_ref.shape[0] * packs
    tp = jnp.dot(tt_ref[...], w_ref[0:h, :], preferred_element_type=jnp.float32)
    sp = jnp.dot(st_ref[...], w_ref[h:2 * h, :], preferred_element_type=jnp.float32)
    stacked = jnp.concatenate([tp, sp], axis=0)
    pieces = []
    for j in range(packs):
        i = lax.broadcasted_iota(jnp.int32, (rows // packs, n_t + n_s), 0)
        i = i * packs + j
        jj = lax.broadcasted_iota(jnp.int32, (rows // packs, n_t + n_s), 1)
        tsel = (jj < n_t) & (i // n_s == jj)
        ssel = (jj >= n_t) & (i % n_s == jj - n_t)
        sel = jnp.where(tsel | ssel, 1.0, 0.0).astype(jnp.float32)
        pieces.append(
            jnp.dot(sel, stacked, preferred_element_type=jnp.float32)
            + b_ref[...])
    out_ref[...] = jnp.concatenate(pieces, axis=1)


def _build_combo(tier_table, scope_table, W, b):
    h = tier_table.shape[1]
    return pl.pallas_call(
        _combo_body,
        out_shape=jax.ShapeDtypeStruct((8, 128), jnp.float32),
    )(tier_table, scope_table, W, b.reshape(1, h))


def _cidx_body(t_ref, s_ref, out_ref, *, n_scope, h, seq):
    cidx = (t_ref[...] * n_scope + s_ref[...]) * h
    k = t_ref.shape[0] // 8
    pieces = []
    for j in range(k):
        blk = cidx[j * 8:(j + 1) * 8, :]
        pieces.append(blk[:, 0:128])
        pieces.append(jnp.pad(blk[:, 128:seq], ((0, 0), (0, 256 - seq))))
    out_ref[...] = jnp.concatenate(pieces, axis=0)


def _build_cidx(tier_2d, scope_2d, n_scope, h):
    bsz, seq = tier_2d.shape
    k = 32
    grid = bsz // (8 * k)
    body = functools.partial(_cidx_body, n_scope=n_scope, h=h, seq=seq)
    return pl.pallas_call(
        body,
        grid=(grid,),
        compiler_params=pltpu.CompilerParams(
            dimension_semantics=("arbitrary",)),
        in_specs=[
            pl.BlockSpec((8 * k, seq), lambda i: (i, 0)),
            pl.BlockSpec((8 * k, seq), lambda i: (i, 0)),
        ],
        out_specs=pl.BlockSpec((16 * k, 128), lambda i: (i, 0)),
        out_shape=jax.ShapeDtypeStruct((bsz * 2, 128), jnp.int32),
    )(tier_2d, scope_2d)


def _sc_lookup(cidx_2d, combo_2d, bsz, seq, h):
    n_pad = cidx_2d.shape[0] * 128
    info = plsc.get_sparse_core_info()
    nc, ns, nl = info.num_cores, info.num_subcores, info.num_lanes
    nw = nc * ns
    tpr = 2048
    ntr = n_pad // tpr
    g = ntr // nw
    ch_out = 8 * seq * h
    assert g * nw == ntr

    plan = []
    for half in range(2):
        for c0 in range(0, 128, nl):
            c = c0 + 128 * half
            if c >= seq:
                continue
            nv = min(nl, seq - c)
            plan.append((half * 1024 + c0, c * h, nv))

    mesh = plsc.VectorSubcoreMesh(core_axis_name="c", subcore_axis_name="s")

    @functools.partial(
        pl.kernel,
        out_type=jax.ShapeDtypeStruct((bsz, seq * h), jnp.float32),
        mesh=mesh,
        compiler_params=pltpu.CompilerParams(needs_layout_passes=False),
        scratch_types=[
            pltpu.VMEM((1024,), jnp.float32),
            pltpu.VMEM((16, 128), jnp.int32),
            pltpu.VMEM((16, 128), jnp.int32),
            pltpu.VMEM((8, seq * h), jnp.float32),
            pltpu.VMEM((8, seq * h), jnp.float32),
            pltpu.SemaphoreType.DMA,
            pltpu.SemaphoreType.DMA,
            pltpu.SemaphoreType.DMA,
            pltpu.SemaphoreType.DMA,
        ],
    )
    def body(cidx_hbm, combo_hbm, out_hbm,
             combo_v, cidx_v0, cidx_v1, rows_v0, rows_v1,
             si0, si1, so0, so1):
        wid = lax.axis_index("s") * nc + lax.axis_index("c")
        tr_base = wid * g
        for i in range(8):
            pltpu.sync_copy(combo_hbm.at[i], combo_v.at[pl.ds(i * 128, 128)])
        iota = lax.iota(jnp.int32, nl)
        cids = (cidx_v0, cidx_v1)
        rowsb = (rows_v0, rows_v1)
        sis = (si0, si1)
        sos = (so0, so1)

        def issue_in(gi, p):
            pltpu.async_copy(
                cidx_hbm.at[pl.ds((tr_base + gi) * 16, 16)], cids[p], sis[p])

        def compute(p):
            def rowloop(rr, c2):
                for (src_c, dst_c, nv) in plan:
                    cvec = cids[p][rr + src_c // 128, pl.ds(src_c % 128, nl)]
                    for j in range(nv):
                        bj = lax.gather(
                            cvec,
                            jnp.full((nl, 1), j, jnp.int32),
                            lax.GatherDimensionNumbers(
                                offset_dims=(),
                                collapsed_slice_dims=(0,),
                                start_index_map=(0,),
                            ),
                            (1,),
                            mode=lax.GatherScatterMode.PROMISE_IN_BOUNDS,
                        )
                        halves = [
                            plsc.load_gather(combo_v, [bj + iota + q * nl])
                            for q in range(h // nl)
                        ]
                        toff = dst_c + j * h
                        for q in range(h // nl):
                            rowsb[p][rr, pl.ds(toff + q * nl, nl)] = halves[q]
                return c2

            lax.fori_loop(0, 8, rowloop, 0)

        issue_in(0, 0)
        npairs = g // 2

        def pair(k, carry):
            for p in (0, 1):
                gi = k * 2 + p
                pltpu.make_async_copy(
                    cidx_hbm.at[pl.ds(0, 16)], cids[p], sis[p]).wait()

                @pl.when(gi + 1 < g)
                def _():
                    issue_in(gi + 1, 1 - p)

                @pl.when(k >= 1)
                def _():
                    pltpu.make_async_copy(
                        rowsb[p], out_hbm.at[pl.ds(0, 8)], sos[p]).wait()

                compute(p)
                pltpu.async_copy(
                    rowsb[p],
                    out_hbm.at[pl.ds((tr_base + gi) * 8, 8)], sos[p])
            return carry

        lax.fori_loop(0, npairs, pair, 0)
        for p in (0, 1):
            pltpu.make_async_copy(
                rowsb[p], out_hbm.at[pl.ds(0, 8)], sos[p]).wait()

    return body(cidx_2d, combo_2d)


def kernel(tier_indices, scope_indices, tier_table, scope_table, W, b):
    bsz, seq = tier_indices.shape
    n_scope = scope_table.shape[0]
    h = tier_table.shape[1]
    combo = _build_combo(tier_table, scope_table, W, b)
    tier_i = tier_indices.astype(jnp.int32)
    scope_i = scope_indices.astype(jnp.int32)
    cidx = _build_cidx(tier_i, scope_i, n_scope, h)
    out = _sc_lookup(cidx, combo, bsz, seq, h)
    return out.reshape(bsz, seq, h)


# speedup vs baseline: 1.2956x; 1.0015x over previous
import functools

import jax
import jax.numpy as jnp
from jax import lax
from jax.experimental import pallas as pl
from jax.experimental.pallas import tpu as pltpu
from jax.experimental.pallas import tpu_sc as plsc


def _combo_body(tt_ref, st_ref, w_ref, b_ref, out_ref):
    h = tt_ref.shape[1]
    n_t = tt_ref.shape[0]
    n_s = st_ref.shape[0]
    lanes = out_ref.shape[1]
    packs = lanes // h
    rows = out_ref.shape[0] * packs
    tp = jnp.dot(tt_ref[...], w_ref[0:h, :], preferred_element_type=jnp.float32)
    sp = jnp.dot(st_ref[...], w_ref[h:2 * h, :], preferred_element_type=jnp.float32)
    stacked = jnp.concatenate([tp, sp], axis=0)
    pieces = []
    for j in range(packs):
        i = lax.broadcasted_iota(jnp.int32, (rows // packs, n_t + n_s), 0)
        i = i * packs + j
        jj = lax.broadcasted_iota(jnp.int32, (rows // packs, n_t + n_s), 1)
        tsel = (jj < n_t) & (i // n_s == jj)
        ssel = (jj >= n_t) & (i % n_s == jj - n_t)
        sel = jnp.where(tsel | ssel, 1.0, 0.0).astype(jnp.float32)
        pieces.append(
            jnp.dot(sel, stacked, preferred_element_type=jnp.float32)
            + b_ref[...])
    out_ref[...] = jnp.concatenate(pieces, axis=1)


def _build_combo(tier_table, scope_table, W, b):
    h = tier_table.shape[1]
    return pl.pallas_call(
        _combo_body,
        out_shape=jax.ShapeDtypeStruct((8, 128), jnp.float32),
    )(tier_table, scope_table, W, b.reshape(1, h))


def _cidx_body(t_ref, s_ref, out_ref, *, n_scope, h, seq):
    cidx = (t_ref[...] * n_scope + s_ref[...]) * h
    k = t_ref.shape[0] // 8
    pieces = []
    for j in range(k):
        blk = cidx[j * 8:(j + 1) * 8, :]
        pieces.append(blk[:, 0:128])
        pieces.append(jnp.pad(blk[:, 128:seq], ((0, 0), (0, 256 - seq))))
    out_ref[...] = jnp.concatenate(pieces, axis=0)


def _build_cidx(tier_2d, scope_2d, n_scope, h):
    bsz, seq = tier_2d.shape
    k = 32
    grid = bsz // (8 * k)
    body = functools.partial(_cidx_body, n_scope=n_scope, h=h, seq=seq)
    return pl.pallas_call(
        body,
        grid=(grid,),
        compiler_params=pltpu.CompilerParams(
            dimension_semantics=("arbitrary",)),
        in_specs=[
            pl.BlockSpec((8 * k, seq), lambda i: (i, 0)),
            pl.BlockSpec((8 * k, seq), lambda i: (i, 0)),
        ],
        out_specs=pl.BlockSpec((16 * k, 128), lambda i: (i, 0)),
        out_shape=jax.ShapeDtypeStruct((bsz * 2, 128), jnp.int32),
    )(tier_2d, scope_2d)


def _sc_lookup(cidx_2d, combo_2d, bsz, seq, h):
    n_pad = cidx_2d.shape[0] * 128
    info = plsc.get_sparse_core_info()
    nc, ns, nl = info.num_cores, info.num_subcores, info.num_lanes
    nw = nc * ns
    tpr = 2048
    ntr = n_pad // tpr
    g = ntr // nw
    ch_out = 8 * seq * h
    assert g * nw == ntr

    plan = []
    for half in range(2):
        for c0 in range(0, 128, nl):
            c = c0 + 128 * half
            if c >= seq:
                continue
            nv = min(nl, seq - c)
            plan.append((half * 1024 + c0, c * h, nv))

    mesh = plsc.VectorSubcoreMesh(core_axis_name="c", subcore_axis_name="s")

    @functools.partial(
        pl.kernel,
        out_type=jax.ShapeDtypeStruct((bsz, seq * h), jnp.float32),
        mesh=mesh,
        compiler_params=pltpu.CompilerParams(needs_layout_passes=False),
        scratch_types=[
            pltpu.VMEM((1024,), jnp.float32),
            pltpu.VMEM((16, 128), jnp.int32),
            pltpu.VMEM((16, 128), jnp.int32),
            pltpu.VMEM((8, seq * h), jnp.float32),
            pltpu.VMEM((8, seq * h), jnp.float32),
            pltpu.SemaphoreType.DMA,
            pltpu.SemaphoreType.DMA,
            pltpu.SemaphoreType.DMA,
            pltpu.SemaphoreType.DMA,
        ],
    )
    def body(cidx_hbm, combo_hbm, out_hbm,
             combo_v, cidx_v0, cidx_v1, rows_v0, rows_v1,
             si0, si1, so0, so1):
        wid = lax.axis_index("s") * nc + lax.axis_index("c")
        tr_base = wid * g
        for i in range(8):
            pltpu.sync_copy(combo_hbm.at[i], combo_v.at[pl.ds(i * 128, 128)])
        iota = lax.iota(jnp.int32, nl)
        cids = (cidx_v0, cidx_v1)
        rowsb = (rows_v0, rows_v1)
        sis = (si0, si1)
        sos = (so0, so1)

        def issue_in(gi, p):
            pltpu.async_copy(
                cidx_hbm.at[pl.ds((tr_base + gi) * 16, 16)], cids[p], sis[p])

        def compute(p):
            @plsc.parallel_loop(0, 8)
            def rowloop(rr):
                for (src_c, dst_c, nv) in plan:
                    cvec = cids[p][rr + src_c // 128, pl.ds(src_c % 128, nl)]
                    for j in range(nv):
                        bj = lax.gather(
                            cvec,
                            jnp.full((nl, 1), j, jnp.int32),
                            lax.GatherDimensionNumbers(
                                offset_dims=(),
                                collapsed_slice_dims=(0,),
                                start_index_map=(0,),
                            ),
                            (1,),
                            mode=lax.GatherScatterMode.PROMISE_IN_BOUNDS,
                        )
                        halves = [
                            plsc.load_gather(combo_v, [bj + iota + q * nl])
                            for q in range(h // nl)
                        ]
                        toff = dst_c + j * h
                        for q in range(h // nl):
                            rowsb[p][rr, pl.ds(toff + q * nl, nl)] = halves[q]

        issue_in(0, 0)
        npairs = g // 2

        def pair(k, carry):
            for p in (0, 1):
                gi = k * 2 + p
                pltpu.make_async_copy(
                    cidx_hbm.at[pl.ds(0, 16)], cids[p], sis[p]).wait()

                @pl.when(gi + 1 < g)
                def _():
                    issue_in(gi + 1, 1 - p)

                @pl.when(k >= 1)
                def _():
                    pltpu.make_async_copy(
                        rowsb[p], out_hbm.at[pl.ds(0, 8)], sos[p]).wait()

                compute(p)
                pltpu.async_copy(
                    rowsb[p],
                    out_hbm.at[pl.ds((tr_base + gi) * 8, 8)], sos[p])
            return carry

        lax.fori_loop(0, npairs, pair, 0)
        for p in (0, 1):
            pltpu.make_async_copy(
                rowsb[p], out_hbm.at[pl.ds(0, 8)], sos[p]).wait()

    return body(cidx_2d, combo_2d)


def kernel(tier_indices, scope_indices, tier_table, scope_table, W, b):
    bsz, seq = tier_indices.shape
    n_scope = scope_table.shape[0]
    h = tier_table.shape[1]
    combo = _build_combo(tier_table, scope_table, W, b)
    tier_i = tier_indices.astype(jnp.int32)
    scope_i = scope_indices.astype(jnp.int32)
    cidx = _build_cidx(tier_i, scope_i, n_scope, h)
    out = _sc_lookup(cidx, combo, bsz, seq, h)
    return out.reshape(bsz, seq, h)
